# baseline (speedup 1.0000x reference)
import functools

import jax
import jax.numpy as jnp
from jax import lax
from jax.experimental import pallas as pl
from jax.experimental.pallas import tpu as pltpu
from jax.experimental.pallas import tpu_sc as plsc

_NUM_CORES = 2
_NUM_SUBCORES = 16
_NUM_WORKERS = _NUM_CORES * _NUM_SUBCORES


def _mesh():
  return plsc.VectorSubcoreMesh(
      core_axis_name="c", subcore_axis_name="s",
      num_cores=_NUM_CORES, num_subcores=_NUM_SUBCORES)


def _worker_id():
  return lax.axis_index("s") * _NUM_CORES + lax.axis_index("c")


def _sc_transpose(table_t, panel=512):
  d, v = table_t.shape
  n_full = v // panel
  tail = v - n_full * panel
  per_w = n_full // _NUM_WORKERS
  n_extra = n_full - per_w * _NUM_WORKERS
  assert per_w % 2 == 1 and n_extra < _NUM_WORKERS

  @functools.partial(
      pl.kernel,
      mesh=_mesh(),
      compiler_params=pltpu.CompilerParams(needs_layout_passes=False),
      out_type=jax.ShapeDtypeStruct((v * d,), jnp.float32),
      scratch_types=[
          pltpu.VMEM((d, panel + 1), jnp.float32),
          pltpu.VMEM((d, panel + 1), jnp.float32),
          pltpu.VMEM((panel * d,), jnp.float32),
          pltpu.VMEM((panel * d,), jnp.float32),
          pltpu.VMEM((d, 64), jnp.float32),
          pltpu.VMEM((64 * d,), jnp.float32),
          pltpu.SemaphoreType.DMA,
      ],
  )
  def k(tab_hbm, out_hbm, in0, in1, o0, o1, tail_v, tail_o, sem):
    wid = _worker_id()
    base = wid * per_w
    iota = lax.iota(jnp.int32, 16)
    iota_hi = iota + 16
    ins = (in0, in1)
    outs = (o0, o1)

    def fire(c, slot):
      pltpu.async_copy(tab_hbm.at[:, pl.ds(c * panel, panel)],
                       ins[slot].at[:, pl.ds(0, panel)], sem)

    def wait(c, slot):
      pltpu.make_async_copy(tab_hbm.at[:, pl.ds(c * panel, panel)],
                            ins[slot].at[:, pl.ds(0, panel)], sem).wait()

    def transpose_slot(slot):
      src = ins[slot]
      dst = outs[slot]

      @plsc.parallel_loop(0, panel, unroll=8)
      def body(b):
        bb = jnp.full((16,), b, jnp.int32)
        dst[pl.ds(b * d, 16)] = plsc.load_gather(src, [iota, bb])
        dst[pl.ds(b * d + 16, 16)] = plsc.load_gather(src, [iota_hi, bb])

    def flush(c, slot):
      pltpu.sync_copy(outs[slot], out_hbm.at[pl.ds(c * panel * d,
                                                   panel * d)])

    fire(base, 0)

    def pair(g, carry):
      c0 = base + 2 * g
      fire(c0 + 1, 1)
      wait(c0, 0)
      transpose_slot(0)
      flush(c0, 0)

      @pl.when(g < per_w // 2 - 1)
      def _():
        fire(c0 + 2, 0)

      wait(c0 + 1, 1)
      transpose_slot(1)
      flush(c0 + 1, 1)
      return carry

    lax.fori_loop(0, per_w // 2, pair, 0)

    c_last = base + per_w - 1
    fire(c_last, 0)
    wait(c_last, 0)
    transpose_slot(0)
    flush(c_last, 0)

    @pl.when(wid < n_extra)
    def _():
      c = _NUM_WORKERS * per_w + wid
      fire(c, 1)
      wait(c, 1)
      transpose_slot(1)
      flush(c, 1)

    if tail:
      @pl.when(wid == _NUM_WORKERS - 1)
      def _():
        col0 = n_full * panel
        pltpu.sync_copy(tab_hbm.at[:, pl.ds(col0, tail)], tail_v)

        def tbody(b, carry):
          bb = jnp.full((16,), b, jnp.int32)
          tail_o[pl.ds(b * d, 16)] = plsc.load_gather(tail_v, [iota, bb])
          tail_o[pl.ds(b * d + 16, 16)] = plsc.load_gather(
              tail_v, [iota_hi, bb])
          return carry

        lax.fori_loop(0, tail, tbody, 0)
        pltpu.sync_copy(tail_o, out_hbm.at[pl.ds(col0 * d, tail * d)])

  return k(table_t)


def _sc_gather(idx_flat, table_lin, chunk=1280):
  n = idx_flat.shape[0]
  v, d = table_lin.shape
  n_per_w = n // _NUM_WORKERS
  n_chunks = n_per_w // chunk
  assert n_per_w % chunk == 0 and n_chunks % 2 == 0

  @functools.partial(
      pl.kernel,
      mesh=_mesh(),
      compiler_params=pltpu.CompilerParams(use_tc_tiling_on_sc=False),
      out_type=jax.ShapeDtypeStruct((n, d), jnp.float32),
      scratch_types=[
          pltpu.VMEM((2, chunk), jnp.int32),
          pltpu.VMEM((2, chunk, d), jnp.float32),
          pltpu.SemaphoreType.DMA,
      ],
  )
  def k(idx_hbm, table_hbm, out_hbm, idx_v, rows_v, sem):
    base = _worker_id() * n_per_w
    n_groups = n_chunks // 2

    def stage_and_fire(c, slot):
      pltpu.sync_copy(idx_hbm.at[pl.ds(base + c * chunk, chunk)],
                      idx_v.at[slot])
      pltpu.async_copy(table_hbm.at[idx_v.at[slot]], rows_v.at[slot], sem)

    def drain_and_flush(c, slot):
      pltpu.make_async_copy(table_hbm.at[idx_v.at[slot]], rows_v.at[slot],
                            sem).wait()
      pltpu.sync_copy(rows_v.at[slot], out_hbm.at[pl.ds(base + c * chunk,
                                                        chunk)])

    stage_and_fire(0, 0)

    def body(g, carry):
      stage_and_fire(2 * g + 1, 1)
      drain_and_flush(2 * g, 0)

      @pl.when(g < n_groups - 1)
      def _():
        stage_and_fire(2 * g + 2, 0)

      drain_and_flush(2 * g + 1, 1)
      return carry

    lax.fori_loop(0, n_groups, body, 0)

  return k(idx_flat, table_lin)


@jax.jit
def _run(input_ids, table):
  b, s = input_ids.shape
  v, d = table.shape
  idx_flat = input_ids.reshape(b * s).astype(jnp.int32)
  table_lin = _sc_transpose(jnp.swapaxes(table, 0, 1)).reshape(v, d)
  out = _sc_gather(idx_flat, table_lin)
  return out.reshape(b, s, d)


def kernel(input_ids, table):
  return _run(input_ids, table)


# speedup vs baseline: 1.0046x; 1.0046x over previous
import functools

import jax
import jax.numpy as jnp
from jax import lax
from jax.experimental import pallas as pl
from jax.experimental.pallas import tpu as pltpu
from jax.experimental.pallas import tpu_sc as plsc

_NUM_CORES = 2
_NUM_SUBCORES = 16
_NUM_WORKERS = _NUM_CORES * _NUM_SUBCORES


def _mesh():
  return plsc.VectorSubcoreMesh(
      core_axis_name="c", subcore_axis_name="s",
      num_cores=_NUM_CORES, num_subcores=_NUM_SUBCORES)


def _worker_id():
  return lax.axis_index("s") * _NUM_CORES + lax.axis_index("c")


def _sc_transpose(table_t, panel=512):
  d, v = table_t.shape
  n_full = v // panel
  tail = v - n_full * panel
  per_w = n_full // _NUM_WORKERS
  n_extra = n_full - per_w * _NUM_WORKERS
  assert per_w % 2 == 1 and n_extra < _NUM_WORKERS

  @functools.partial(
      pl.kernel,
      mesh=_mesh(),
      compiler_params=pltpu.CompilerParams(needs_layout_passes=False),
      out_type=jax.ShapeDtypeStruct((v * d,), jnp.float32),
      scratch_types=[
          pltpu.VMEM((d, panel + 1), jnp.float32),
          pltpu.VMEM((d, panel + 1), jnp.float32),
          pltpu.VMEM((panel * d,), jnp.float32),
          pltpu.VMEM((panel * d,), jnp.float32),
          pltpu.VMEM((d, 64), jnp.float32),
          pltpu.VMEM((64 * d,), jnp.float32),
          pltpu.SemaphoreType.DMA,
      ],
  )
  def k(tab_hbm, out_hbm, in0, in1, o0, o1, tail_v, tail_o, sem):
    wid = _worker_id()
    base = wid * per_w
    iota = lax.iota(jnp.int32, 16)
    iota_hi = iota + 16
    ins = (in0, in1)
    outs = (o0, o1)

    def fire(c, slot):
      pltpu.async_copy(tab_hbm.at[:, pl.ds(c * panel, panel)],
                       ins[slot].at[:, pl.ds(0, panel)], sem)

    def wait(c, slot):
      pltpu.make_async_copy(tab_hbm.at[:, pl.ds(c * panel, panel)],
                            ins[slot].at[:, pl.ds(0, panel)], sem).wait()

    def transpose_slot(slot):
      src = ins[slot]
      dst = outs[slot]

      @plsc.parallel_loop(0, panel, step=8, unroll=4)
      def body(b0):
        bb0 = jnp.full((16,), b0, jnp.int32)
        o0 = b0 * d
        for kk in range(8):
          bb = bb0 + kk
          dst[pl.ds(o0 + kk * d, 16)] = plsc.load_gather(src, [iota, bb])
          dst[pl.ds(o0 + kk * d + 16, 16)] = plsc.load_gather(
              src, [iota_hi, bb])

    def flush(c, slot):
      pltpu.sync_copy(outs[slot], out_hbm.at[pl.ds(c * panel * d,
                                                   panel * d)])

    fire(base, 0)

    def pair(g, carry):
      c0 = base + 2 * g
      fire(c0 + 1, 1)
      wait(c0, 0)
      transpose_slot(0)
      flush(c0, 0)

      @pl.when(g < per_w // 2 - 1)
      def _():
        fire(c0 + 2, 0)

      wait(c0 + 1, 1)
      transpose_slot(1)
      flush(c0 + 1, 1)
      return carry

    lax.fori_loop(0, per_w // 2, pair, 0)

    c_last = base + per_w - 1
    fire(c_last, 0)
    wait(c_last, 0)
    transpose_slot(0)
    flush(c_last, 0)

    @pl.when(wid < n_extra)
    def _():
      c = _NUM_WORKERS * per_w + wid
      fire(c, 1)
      wait(c, 1)
      transpose_slot(1)
      flush(c, 1)

    if tail:
      @pl.when(wid == _NUM_WORKERS - 1)
      def _():
        col0 = n_full * panel
        pltpu.sync_copy(tab_hbm.at[:, pl.ds(col0, tail)], tail_v)

        def tbody(b, carry):
          bb = jnp.full((16,), b, jnp.int32)
          tail_o[pl.ds(b * d, 16)] = plsc.load_gather(tail_v, [iota, bb])
          tail_o[pl.ds(b * d + 16, 16)] = plsc.load_gather(
              tail_v, [iota_hi, bb])
          return carry

        lax.fori_loop(0, tail, tbody, 0)
        pltpu.sync_copy(tail_o, out_hbm.at[pl.ds(col0 * d, tail * d)])

  return k(table_t)


def _sc_gather(idx_flat, table_lin, chunk=1280):
  n = idx_flat.shape[0]
  v, d = table_lin.shape
  n_per_w = n // _NUM_WORKERS
  n_chunks = n_per_w // chunk
  assert n_per_w % chunk == 0 and n_chunks % 2 == 0

  @functools.partial(
      pl.kernel,
      mesh=_mesh(),
      compiler_params=pltpu.CompilerParams(use_tc_tiling_on_sc=False),
      out_type=jax.ShapeDtypeStruct((n, d), jnp.float32),
      scratch_types=[
          pltpu.VMEM((2, chunk), jnp.int32),
          pltpu.VMEM((2, chunk, d), jnp.float32),
          pltpu.SemaphoreType.DMA,
      ],
  )
  def k(idx_hbm, table_hbm, out_hbm, idx_v, rows_v, sem):
    base = _worker_id() * n_per_w
    n_groups = n_chunks // 2

    def stage_and_fire(c, slot):
      pltpu.sync_copy(idx_hbm.at[pl.ds(base + c * chunk, chunk)],
                      idx_v.at[slot])
      pltpu.async_copy(table_hbm.at[idx_v.at[slot]], rows_v.at[slot], sem)

    def drain_and_flush(c, slot):
      pltpu.make_async_copy(table_hbm.at[idx_v.at[slot]], rows_v.at[slot],
                            sem).wait()
      pltpu.sync_copy(rows_v.at[slot], out_hbm.at[pl.ds(base + c * chunk,
                                                        chunk)])

    stage_and_fire(0, 0)

    def body(g, carry):
      stage_and_fire(2 * g + 1, 1)
      drain_and_flush(2 * g, 0)

      @pl.when(g < n_groups - 1)
      def _():
        stage_and_fire(2 * g + 2, 0)

      drain_and_flush(2 * g + 1, 1)
      return carry

    lax.fori_loop(0, n_groups, body, 0)

  return k(idx_flat, table_lin)


@jax.jit
def _run(input_ids, table):
  b, s = input_ids.shape
  v, d = table.shape
  idx_flat = input_ids.reshape(b * s).astype(jnp.int32)
  table_lin = _sc_transpose(jnp.swapaxes(table, 0, 1)).reshape(v, d)
  out = _sc_gather(idx_flat, table_lin)
  return out.reshape(b, s, d)


def kernel(input_ids, table):
  return _run(input_ids, table)


# speedup vs baseline: 1.5252x; 1.5183x over previous
import functools

import jax
import jax.numpy as jnp
from jax import lax
from jax.experimental import pallas as pl
from jax.experimental.pallas import tpu as pltpu
from jax.experimental.pallas import tpu_sc as plsc

_NUM_CORES = 2
_NUM_SUBCORES = 16
_NUM_WORKERS = _NUM_CORES * _NUM_SUBCORES


def _mesh():
  return plsc.VectorSubcoreMesh(
      core_axis_name="c", subcore_axis_name="s",
      num_cores=_NUM_CORES, num_subcores=_NUM_SUBCORES)


def _worker_id():
  return lax.axis_index("s") * _NUM_CORES + lax.axis_index("c")


def _sc_transpose(table_t, panel=512):
  d, v = table_t.shape
  n_full = v // panel
  tail = v - n_full * panel
  per_w = n_full // _NUM_WORKERS
  n_extra = n_full - per_w * _NUM_WORKERS
  assert per_w % 2 == 1 and n_extra < _NUM_WORKERS

  @functools.partial(
      pl.kernel,
      mesh=_mesh(),
      compiler_params=pltpu.CompilerParams(needs_layout_passes=False),
      out_type=jax.ShapeDtypeStruct((v * d,), jnp.float32),
      scratch_types=[
          pltpu.VMEM((d, panel + 1), jnp.float32),
          pltpu.VMEM((d, panel + 1), jnp.float32),
          pltpu.VMEM((panel * d,), jnp.float32),
          pltpu.VMEM((panel * d,), jnp.float32),
          pltpu.VMEM((d, 64), jnp.float32),
          pltpu.VMEM((64 * d,), jnp.float32),
          pltpu.SemaphoreType.DMA,
      ],
  )
  def k(tab_hbm, out_hbm, in0, in1, o0, o1, tail_v, tail_o, sem):
    wid = _worker_id()
    base = wid * per_w
    iota = lax.iota(jnp.int32, 16)
    iota_hi = iota + 16
    ins = (in0, in1)
    outs = (o0, o1)

    def fire(c, slot):
      pltpu.async_copy(tab_hbm.at[:, pl.ds(c * panel, panel)],
                       ins[slot].at[:, pl.ds(0, panel)], sem)

    def wait(c, slot):
      pltpu.make_async_copy(tab_hbm.at[:, pl.ds(c * panel, panel)],
                            ins[slot].at[:, pl.ds(0, panel)], sem).wait()

    def transpose_slot(slot):
      src = ins[slot]
      dst = outs[slot]

      @plsc.parallel_loop(0, panel, step=8, unroll=4)
      def body(b0):
        bb0 = jnp.full((16,), b0, jnp.int32)
        o0 = b0 * d
        for kk in range(8):
          bb = bb0 + kk
          dst[pl.ds(o0 + kk * d, 16)] = plsc.load_gather(src, [iota, bb])
          dst[pl.ds(o0 + kk * d + 16, 16)] = plsc.load_gather(
              src, [iota_hi, bb])

    def flush(c, slot):
      pltpu.sync_copy(outs[slot], out_hbm.at[pl.ds(c * panel * d,
                                                   panel * d)])

    fire(base, 0)

    def pair(g, carry):
      c0 = base + 2 * g
      fire(c0 + 1, 1)
      wait(c0, 0)
      transpose_slot(0)
      flush(c0, 0)

      @pl.when(g < per_w // 2 - 1)
      def _():
        fire(c0 + 2, 0)

      wait(c0 + 1, 1)
      transpose_slot(1)
      flush(c0 + 1, 1)
      return carry

    lax.fori_loop(0, per_w // 2, pair, 0)

    c_last = base + per_w - 1
    fire(c_last, 0)
    wait(c_last, 0)
    transpose_slot(0)
    flush(c_last, 0)

    @pl.when(wid < n_extra)
    def _():
      c = _NUM_WORKERS * per_w + wid
      fire(c, 1)
      wait(c, 1)
      transpose_slot(1)
      flush(c, 1)

    if tail:
      @pl.when(wid == _NUM_WORKERS - 1)
      def _():
        col0 = n_full * panel
        pltpu.sync_copy(tab_hbm.at[:, pl.ds(col0, tail)], tail_v)

        def tbody(b, carry):
          bb = jnp.full((16,), b, jnp.int32)
          tail_o[pl.ds(b * d, 16)] = plsc.load_gather(tail_v, [iota, bb])
          tail_o[pl.ds(b * d + 16, 16)] = plsc.load_gather(
              tail_v, [iota_hi, bb])
          return carry

        lax.fori_loop(0, tail, tbody, 0)
        pltpu.sync_copy(tail_o, out_hbm.at[pl.ds(col0 * d, tail * d)])

  return k(table_t)


def _sc_gather(idx_flat, table_lin, chunk=1280):
  n = idx_flat.shape[0]
  v, d = table_lin.shape
  n_per_w = n // _NUM_WORKERS
  n_chunks = n_per_w // chunk
  assert n_per_w % chunk == 0 and n_chunks % 2 == 0

  @functools.partial(
      pl.kernel,
      mesh=_mesh(),
      compiler_params=pltpu.CompilerParams(use_tc_tiling_on_sc=False),
      out_type=jax.ShapeDtypeStruct((n, d), jnp.float32),
      scratch_types=[
          pltpu.VMEM((2, chunk), jnp.int32),
          pltpu.VMEM((2, chunk, d), jnp.float32),
          pltpu.SemaphoreType.DMA,
      ],
  )
  def k(idx_hbm, table_hbm, out_hbm, idx_v, rows_v, sem):
    base = _worker_id() * n_per_w
    n_groups = n_chunks // 2

    def stage_and_fire(c, slot):
      pltpu.sync_copy(idx_hbm.at[pl.ds(base + c * chunk, chunk)],
                      idx_v.at[slot])
      pltpu.async_copy(table_hbm.at[idx_v.at[slot]], rows_v.at[slot], sem)

    def drain_and_flush(c, slot):
      pltpu.make_async_copy(table_hbm.at[idx_v.at[slot]], rows_v.at[slot],
                            sem).wait()
      pltpu.sync_copy(rows_v.at[slot], out_hbm.at[pl.ds(base + c * chunk,
                                                        chunk)])

    stage_and_fire(0, 0)

    def body(g, carry):
      stage_and_fire(2 * g + 1, 1)
      drain_and_flush(2 * g, 0)

      @pl.when(g < n_groups - 1)
      def _():
        stage_and_fire(2 * g + 2, 0)

      drain_and_flush(2 * g + 1, 1)
      return carry

    lax.fori_loop(0, n_groups, body, 0)

  return k(idx_flat, table_lin)


def _tc_format(rows128, b, s, d):
  bt = 128
  g = d * bt // 128
  assert (bt * s * d) % 128 == 0

  def body(in_ref, out_ref):
    x = in_ref[...].reshape(bt, s * d)
    out_ref[...] = x.T.reshape(s, d, bt)

  return pl.pallas_call(
      body,
      grid=(b // bt,),
      in_specs=[pl.BlockSpec((bt * s * d // 128, 128), lambda i: (i, 0))],
      out_specs=pl.BlockSpec((s, d, bt), lambda i: (0, 0, i)),
      out_shape=jax.ShapeDtypeStruct((s, d, b), jnp.float32),
  )(rows128)


@jax.jit
def _run(input_ids, table):
  b, s = input_ids.shape
  v, d = table.shape
  idx_flat = input_ids.reshape(b * s).astype(jnp.int32)
  table_lin = _sc_transpose(jnp.swapaxes(table, 0, 1)).reshape(v, d)
  out = _sc_gather(idx_flat, table_lin)
  out_t = _tc_format(out.reshape(b * s * d // 128, 128), b, s, d)
  return jnp.transpose(out_t, (2, 0, 1))


def kernel(input_ids, table):
  return _run(input_ids, table)
